# Initial kernel scaffold; baseline (speedup 1.0000x reference)
import jax
import jax.numpy as jnp
from jax.experimental import pallas as pl


def kernel(x, edge_index, edge_attr, W1, b1, W2, b2, W_ih, b_ih, W_hh, b_hh):
    raise NotImplementedError("write your pallas kernel here")


# speedup vs baseline: 4.2211x; 4.2211x over previous
import functools

import jax
import jax.numpy as jnp
from jax import lax
from jax.experimental import pallas as pl
from jax.experimental.pallas import tpu as pltpu
from jax.experimental.pallas import tpu_sc as plsc

N = 100000
E = 1600000
D_EDGE = 16
D_HID1 = 16
MSG = 8
HID = 8

NUM_CORES = 2
NUM_SUBCORES = 16
NUM_WORKERS = NUM_CORES * NUM_SUBCORES
LANE = 128
INNER = 8
OUTER = 49
BLOCKS_PER_WORKER = INNER * OUTER
CHUNK = INNER * LANE
E_PAD = NUM_WORKERS * BLOCKS_PER_WORKER * LANE
NUM_BLOCKS = E_PAD // LANE
PAD = E_PAD - E
N_DUMMY = 480
N_ACC = N + N_DUMMY
ROWS_PER_TILE = N_ACC // NUM_SUBCORES

_MESH = plsc.VectorSubcoreMesh(
    core_axis_name="c", subcore_axis_name="s",
    num_cores=NUM_CORES, num_subcores=NUM_SUBCORES)


@functools.partial(
    pl.kernel,
    out_type=jax.ShapeDtypeStruct((E_PAD, HID), jnp.float32),
    mesh=_MESH,
    scratch_types=[
        pltpu.VMEM((INNER, LANE), jnp.int32),
        pltpu.VMEM((CHUNK, HID), jnp.float32),
        pltpu.SemaphoreType.DMA,
    ],
    compiler_params=pltpu.CompilerParams(use_tc_tiling_on_sc=False),
)
def _sc_gather(x_hbm, src_hbm, out_hbm, idx_v, rows_v, sem):
    wid = lax.axis_index("s") * NUM_CORES + lax.axis_index("c")

    def body(i, carry):
        blk = wid * BLOCKS_PER_WORKER + i * INNER
        pltpu.sync_copy(src_hbm.at[pl.ds(blk, INNER)], idx_v)
        cps = [
            pltpu.async_copy(
                x_hbm.at[idx_v.at[j]],
                rows_v.at[pl.ds(j * LANE, LANE)], sem)
            for j in range(INNER)
        ]
        for cp in cps:
            cp.wait()
        pltpu.sync_copy(rows_v, out_hbm.at[pl.ds(blk * LANE, CHUNK)])
        return carry

    lax.fori_loop(0, OUTER, body, 0)


@functools.partial(
    pl.kernel,
    out_type=jax.ShapeDtypeStruct((NUM_CORES, N_ACC, HID), jnp.float32),
    mesh=_MESH,
    scratch_types=[
        pltpu.VMEM((INNER, LANE), jnp.int32),
        pltpu.VMEM((CHUNK, HID), jnp.float32),
        pltpu.VMEM_SHARED((N_ACC, HID), jnp.float32),
        pltpu.SemaphoreType.DMA,
    ],
    compiler_params=pltpu.CompilerParams(use_tc_tiling_on_sc=False),
)
def _sc_scatter(msg_hbm, dst_hbm, zero_hbm, out_hbm, idx_v, rows_v, acc, sem):
    c = lax.axis_index("c")
    s = lax.axis_index("s")
    wid = s * NUM_CORES + c
    t0 = s * ROWS_PER_TILE
    pltpu.sync_copy(zero_hbm.at[pl.ds(t0, ROWS_PER_TILE)],
                    acc.at[pl.ds(t0, ROWS_PER_TILE)])
    plsc.subcore_barrier()

    def body(i, carry):
        blk = wid * BLOCKS_PER_WORKER + i * INNER
        pltpu.sync_copy(dst_hbm.at[pl.ds(blk, INNER)], idx_v)
        pltpu.sync_copy(msg_hbm.at[pl.ds(blk * LANE, CHUNK)], rows_v)
        for j in range(INNER):
            pltpu.sync_copy(rows_v.at[pl.ds(j * LANE, LANE)],
                            acc.at[idx_v.at[j]], add=True)
        return carry

    lax.fori_loop(0, OUTER, body, 0)
    plsc.subcore_barrier()
    pltpu.sync_copy(acc.at[pl.ds(t0, ROWS_PER_TILE)],
                    out_hbm.at[c].at[pl.ds(t0, ROWS_PER_TILE)])


T_EDGE = 8000


def _edge_body(ea_ref, hs_ref, w1_ref, b1_ref, w2_ref, b2_ref, p_ref, s_ref,
               msg_ref):
    f32 = jnp.float32
    e1 = jnp.maximum(
        jnp.dot(ea_ref[...], w1_ref[...], preferred_element_type=f32)
        + b1_ref[...], 0.0)
    e2 = jnp.maximum(
        jnp.dot(e1, w2_ref[...], preferred_element_type=f32)
        + b2_ref[...], 0.0)
    hst = jnp.dot(hs_ref[...], p_ref[...], preferred_element_type=f32)
    msg_ref[...] = jnp.dot(e2 * hst, s_ref[...], preferred_element_type=f32)


_edge_call = pl.pallas_call(
    _edge_body,
    grid=(E // T_EDGE,),
    in_specs=[
        pl.BlockSpec((T_EDGE, D_EDGE), lambda i: (i, 0)),
        pl.BlockSpec((T_EDGE, HID), lambda i: (i, 0)),
        pl.BlockSpec((D_EDGE, D_HID1), lambda i: (0, 0)),
        pl.BlockSpec((1, D_HID1), lambda i: (0, 0)),
        pl.BlockSpec((D_HID1, MSG * HID), lambda i: (0, 0)),
        pl.BlockSpec((1, MSG * HID), lambda i: (0, 0)),
        pl.BlockSpec((HID, MSG * HID), lambda i: (0, 0)),
        pl.BlockSpec((MSG * HID, MSG), lambda i: (0, 0)),
    ],
    out_specs=pl.BlockSpec((T_EDGE, MSG), lambda i: (i, 0)),
    out_shape=jax.ShapeDtypeStruct((E_PAD, MSG), jnp.float32),
)


T_NODE = 2000


def _gru_body(x_ref, p_ref, wir, wiz, win, whr, whz, whn,
              bir, biz, bin_, bhr, bhz, bhn, out_ref):
    f32 = jnp.float32
    m = p_ref[0] + p_ref[1]
    x = x_ref[...]
    r = jax.nn.sigmoid(
        jnp.dot(m, wir[...], preferred_element_type=f32) + bir[...]
        + jnp.dot(x, whr[...], preferred_element_type=f32) + bhr[...])
    z = jax.nn.sigmoid(
        jnp.dot(m, wiz[...], preferred_element_type=f32) + biz[...]
        + jnp.dot(x, whz[...], preferred_element_type=f32) + bhz[...])
    n = jnp.tanh(
        jnp.dot(m, win[...], preferred_element_type=f32) + bin_[...]
        + r * (jnp.dot(x, whn[...], preferred_element_type=f32) + bhn[...]))
    out_ref[...] = (1.0 - z) * n + z * x


_w_spec = pl.BlockSpec((HID, HID), lambda i: (0, 0))
_b_spec = pl.BlockSpec((1, HID), lambda i: (0, 0))
_gru_call = pl.pallas_call(
    _gru_body,
    grid=(N // T_NODE,),
    in_specs=[
        pl.BlockSpec((T_NODE, HID), lambda i: (i, 0)),
        pl.BlockSpec((NUM_CORES, T_NODE, HID), lambda i: (0, i, 0)),
        _w_spec, _w_spec, _w_spec, _w_spec, _w_spec, _w_spec,
        _b_spec, _b_spec, _b_spec, _b_spec, _b_spec, _b_spec,
    ],
    out_specs=pl.BlockSpec((T_NODE, HID), lambda i: (i, 0)),
    out_shape=jax.ShapeDtypeStruct((N, HID), jnp.float32),
)


def kernel(x, edge_index, edge_attr, W1, b1, W2, b2, W_ih, b_ih, W_hh, b_hh):
    src = edge_index[0]
    dst = edge_index[1]
    pad = jnp.arange(PAD, dtype=jnp.int32)
    src_p = jnp.concatenate([src, pad]).reshape(NUM_BLOCKS, LANE)
    dst_p = jnp.concatenate(
        [dst, jnp.int32(N) + (pad % N_DUMMY)]).reshape(NUM_BLOCKS, LANE)

    h_src = _sc_gather(x, src_p)

    P = jnp.tile(jnp.eye(HID, dtype=jnp.float32), (1, MSG))
    S = jnp.repeat(jnp.eye(MSG, dtype=jnp.float32), HID, axis=0)
    msg = _edge_call(edge_attr, h_src, W1, b1.reshape(1, -1), W2,
                     b2.reshape(1, -1), P, S)

    zero_acc = jnp.zeros((N_ACC, HID), jnp.float32)
    partial = _sc_scatter(msg, dst_p, zero_acc)

    h_new = _gru_call(
        x, partial,
        W_ih[:, 0:HID], W_ih[:, HID:2 * HID], W_ih[:, 2 * HID:],
        W_hh[:, 0:HID], W_hh[:, HID:2 * HID], W_hh[:, 2 * HID:],
        b_ih[0:HID].reshape(1, -1), b_ih[HID:2 * HID].reshape(1, -1),
        b_ih[2 * HID:].reshape(1, -1),
        b_hh[0:HID].reshape(1, -1), b_hh[HID:2 * HID].reshape(1, -1),
        b_hh[2 * HID:].reshape(1, -1),
    )
    return h_new


# speedup vs baseline: 4.6995x; 1.1133x over previous
import functools

import jax
import jax.numpy as jnp
from jax import lax
from jax.experimental import pallas as pl
from jax.experimental.pallas import tpu as pltpu
from jax.experimental.pallas import tpu_sc as plsc

N = 100000
E = 1600000
D_EDGE = 16
D_HID1 = 16
MSG = 8
HID = 8

NUM_CORES = 2
NUM_SUBCORES = 16
NUM_WORKERS = NUM_CORES * NUM_SUBCORES
LANE = 128
INNER = 8
OUTER = 49
BLOCKS_PER_WORKER = INNER * OUTER
CHUNK = INNER * LANE
E_PAD = NUM_WORKERS * BLOCKS_PER_WORKER * LANE
NUM_BLOCKS = E_PAD // LANE
PAD = E_PAD - E
N_DUMMY = 480
N_ACC = N + N_DUMMY
ROWS_PER_TILE = N_ACC // NUM_SUBCORES

_MESH = plsc.VectorSubcoreMesh(
    core_axis_name="c", subcore_axis_name="s",
    num_cores=NUM_CORES, num_subcores=NUM_SUBCORES)


@functools.partial(
    pl.kernel,
    out_type=jax.ShapeDtypeStruct((HID, E_PAD), jnp.float32),
    mesh=_MESH,
    scratch_types=[
        pltpu.VMEM((INNER, LANE), jnp.int32),
        pltpu.VMEM((CHUNK, HID), jnp.float32),
        pltpu.VMEM((HID, CHUNK), jnp.float32),
        pltpu.SemaphoreType.DMA,
    ],
    compiler_params=pltpu.CompilerParams(use_tc_tiling_on_sc=False, needs_layout_passes=False),
)
def _sc_gather(x_hbm, src_hbm, out_hbm, idx_v, rows_v, cols_v, sem):
    wid = lax.axis_index("s") * NUM_CORES + lax.axis_index("c")
    iota = lax.iota(jnp.int32, 16)

    def body(i, carry):
        blk = wid * BLOCKS_PER_WORKER + i * INNER
        pltpu.sync_copy(src_hbm.at[pl.ds(blk, INNER)], idx_v)
        cps = [
            pltpu.async_copy(
                x_hbm.at[idx_v.at[j]],
                rows_v.at[pl.ds(j * LANE, LANE)], sem)
            for j in range(INNER)
        ]
        for cp in cps:
            cp.wait()

        def repack(g, carry2):
            rows16 = g * 16 + iota
            for k in range(HID):
                vals = plsc.load_gather(
                    rows_v, [rows16, jnp.full((16,), k, jnp.int32)])
                cols_v[k, pl.ds(g * 16, 16)] = vals
            return carry2

        lax.fori_loop(0, CHUNK // 16, repack, 0)
        for k in range(HID):
            pltpu.sync_copy(cols_v.at[k],
                            out_hbm.at[k, pl.ds(blk * LANE, CHUNK)])
        return carry

    lax.fori_loop(0, OUTER, body, 0)


@functools.partial(
    pl.kernel,
    out_type=jax.ShapeDtypeStruct((NUM_CORES, N_ACC, HID), jnp.float32),
    mesh=_MESH,
    scratch_types=[
        pltpu.VMEM((INNER, LANE), jnp.int32),
        pltpu.VMEM((CHUNK, HID), jnp.float32),
        pltpu.VMEM((HID, CHUNK), jnp.float32),
        pltpu.VMEM_SHARED((N_ACC, HID), jnp.float32),
        pltpu.SemaphoreType.DMA,
    ],
    compiler_params=pltpu.CompilerParams(use_tc_tiling_on_sc=False, needs_layout_passes=False),
)
def _sc_scatter(msg_hbm, dst_hbm, zero_hbm, out_hbm, idx_v, rows_v, cols_v,
                acc, sem):
    c = lax.axis_index("c")
    s = lax.axis_index("s")
    wid = s * NUM_CORES + c
    t0 = s * ROWS_PER_TILE
    iota = lax.iota(jnp.int32, 16)
    pltpu.sync_copy(zero_hbm.at[pl.ds(t0, ROWS_PER_TILE)],
                    acc.at[pl.ds(t0, ROWS_PER_TILE)])
    plsc.subcore_barrier()

    def body(i, carry):
        blk = wid * BLOCKS_PER_WORKER + i * INNER
        pltpu.sync_copy(dst_hbm.at[pl.ds(blk, INNER)], idx_v)
        for k in range(HID):
            pltpu.sync_copy(msg_hbm.at[k, pl.ds(blk * LANE, CHUNK)],
                            cols_v.at[k])

        def repack(g, carry2):
            rows16 = g * 16 + iota
            for k in range(HID):
                vals = cols_v[k, pl.ds(g * 16, 16)]
                plsc.store_scatter(
                    rows_v, [rows16, jnp.full((16,), k, jnp.int32)], vals)
            return carry2

        lax.fori_loop(0, CHUNK // 16, repack, 0)
        for j in range(INNER):
            pltpu.sync_copy(rows_v.at[pl.ds(j * LANE, LANE)],
                            acc.at[idx_v.at[j]], add=True)
        return carry

    lax.fori_loop(0, OUTER, body, 0)
    plsc.subcore_barrier()
    pltpu.sync_copy(acc.at[pl.ds(t0, ROWS_PER_TILE)],
                    out_hbm.at[c].at[pl.ds(t0, ROWS_PER_TILE)])


T_EDGE = 12800


def _edge_body(eat_ref, hst_ref, w1t_ref, b1c_ref, w2t_ref, b2c_ref,
               pt_ref, st_ref, msgt_ref):
    f32 = jnp.float32
    e1 = jnp.maximum(
        jnp.dot(w1t_ref[...], eat_ref[...], preferred_element_type=f32)
        + b1c_ref[...], 0.0)
    e2 = jnp.maximum(
        jnp.dot(w2t_ref[...], e1, preferred_element_type=f32)
        + b2c_ref[...], 0.0)
    hst = jnp.dot(pt_ref[...], hst_ref[...], preferred_element_type=f32)
    msgt_ref[...] = jnp.dot(st_ref[...], e2 * hst,
                            preferred_element_type=f32)


_edge_call = pl.pallas_call(
    _edge_body,
    grid=(E // T_EDGE,),
    in_specs=[
        pl.BlockSpec((D_EDGE, T_EDGE), lambda i: (0, i)),
        pl.BlockSpec((HID, T_EDGE), lambda i: (0, i)),
        pl.BlockSpec((D_EDGE, D_HID1), lambda i: (0, 0)),
        pl.BlockSpec((D_HID1, 1), lambda i: (0, 0)),
        pl.BlockSpec((MSG * HID, D_HID1), lambda i: (0, 0)),
        pl.BlockSpec((MSG * HID, 1), lambda i: (0, 0)),
        pl.BlockSpec((MSG * HID, HID), lambda i: (0, 0)),
        pl.BlockSpec((MSG, MSG * HID), lambda i: (0, 0)),
    ],
    out_specs=pl.BlockSpec((MSG, T_EDGE), lambda i: (0, i)),
    out_shape=jax.ShapeDtypeStruct((MSG, E_PAD), jnp.float32),
)


NP = N // 16
NP_ACC = N_ACC // 16


def _gru_body(x_ref, p_ref, wir, wiz, win, whr, whz, whn,
              bir, biz, bin_, bhr, bhz, bhn, out_ref):
    f32 = jnp.float32
    m = p_ref[0] + p_ref[1]
    x = x_ref[...]
    r = jax.nn.sigmoid(
        jnp.dot(m, wir[...], preferred_element_type=f32) + bir[...]
        + jnp.dot(x, whr[...], preferred_element_type=f32) + bhr[...])
    z = jax.nn.sigmoid(
        jnp.dot(m, wiz[...], preferred_element_type=f32) + biz[...]
        + jnp.dot(x, whz[...], preferred_element_type=f32) + bhz[...])
    n = jnp.tanh(
        jnp.dot(m, win[...], preferred_element_type=f32) + bin_[...]
        + r * (jnp.dot(x, whn[...], preferred_element_type=f32) + bhn[...]))
    out_ref[...] = (1.0 - z) * n + z * x


_wp_spec = pl.BlockSpec((16 * HID, 16 * HID), lambda i: (0, 0))
_bp_spec = pl.BlockSpec((1, 16 * HID), lambda i: (0, 0))
_gru_call = pl.pallas_call(
    _gru_body,
    grid=(1,),
    in_specs=[
        pl.BlockSpec((NP, 16 * HID), lambda i: (0, 0)),
        pl.BlockSpec((NUM_CORES, NP, 16 * HID), lambda i: (0, 0, 0)),
        _wp_spec, _wp_spec, _wp_spec, _wp_spec, _wp_spec, _wp_spec,
        _bp_spec, _bp_spec, _bp_spec, _bp_spec, _bp_spec, _bp_spec,
    ],
    out_specs=pl.BlockSpec((NP, 16 * HID), lambda i: (0, 0)),
    out_shape=jax.ShapeDtypeStruct((NP, 16 * HID), jnp.float32),
)


def _bd(w):
    return jnp.kron(jnp.eye(16, dtype=jnp.float32), w)


def _bp(b):
    return jnp.tile(b, 16).reshape(1, 16 * HID)


def kernel(x, edge_index, edge_attr, W1, b1, W2, b2, W_ih, b_ih, W_hh, b_hh):
    src = edge_index[0]
    dst = edge_index[1]
    pad = jnp.arange(PAD, dtype=jnp.int32)
    src_p = jnp.concatenate([src, pad]).reshape(NUM_BLOCKS, LANE)
    dst_p = jnp.concatenate(
        [dst, jnp.int32(N) + (pad % N_DUMMY)]).reshape(NUM_BLOCKS, LANE)

    h_srcT = _sc_gather(x, src_p)

    P = jnp.tile(jnp.eye(HID, dtype=jnp.float32), (1, MSG))
    S = jnp.repeat(jnp.eye(MSG, dtype=jnp.float32), HID, axis=0)
    msgT = _edge_call(edge_attr.T, h_srcT, W1.T, b1.reshape(-1, 1), W2.T,
                      b2.reshape(-1, 1), P.T, S.T)

    zero_acc = jnp.zeros((N_ACC, HID), jnp.float32)
    partial = _sc_scatter(msgT, dst_p, zero_acc)

    x_p = x.reshape(NP, 16 * HID)
    partial_p = partial.reshape(NUM_CORES, NP_ACC, 16 * HID)[:, :NP]
    h_new_p = _gru_call(
        x_p, partial_p,
        _bd(W_ih[:, 0:HID]), _bd(W_ih[:, HID:2 * HID]),
        _bd(W_ih[:, 2 * HID:]),
        _bd(W_hh[:, 0:HID]), _bd(W_hh[:, HID:2 * HID]),
        _bd(W_hh[:, 2 * HID:]),
        _bp(b_ih[0:HID]), _bp(b_ih[HID:2 * HID]), _bp(b_ih[2 * HID:]),
        _bp(b_hh[0:HID]), _bp(b_hh[HID:2 * HID]), _bp(b_hh[2 * HID:]),
    )
    return h_new_p.reshape(N, HID)


# speedup vs baseline: 13.2504x; 2.8195x over previous
import functools

import jax
import jax.numpy as jnp
from jax import lax
from jax.experimental import pallas as pl
from jax.experimental.pallas import tpu as pltpu
from jax.experimental.pallas import tpu_sc as plsc

N = 100000
E = 1600000
D_EDGE = 16
D_HID1 = 16
MSG = 8
HID = 8

NUM_CORES = 2
NUM_SUBCORES = 16
NUM_WORKERS = NUM_CORES * NUM_SUBCORES
LANE = 128
INNER = 8
OUTER = 49
BLOCKS_PER_WORKER = INNER * OUTER
CHUNK = INNER * LANE
E_PAD = NUM_WORKERS * BLOCKS_PER_WORKER * LANE
NUM_BLOCKS = E_PAD // LANE
PAD = E_PAD - E
N_DUMMY = 608
N_ACC = N + N_DUMMY
ROWS_PER_TILE = N_ACC // NUM_SUBCORES

_MESH = plsc.VectorSubcoreMesh(
    core_axis_name="c", subcore_axis_name="s",
    num_cores=NUM_CORES, num_subcores=NUM_SUBCORES)
_SC_PARAMS = pltpu.CompilerParams(
    use_tc_tiling_on_sc=False, needs_layout_passes=False)


@functools.partial(
    pl.kernel,
    out_type=jax.ShapeDtypeStruct((NUM_BLOCKS, HID, LANE), jnp.float32),
    mesh=_MESH,
    scratch_types=[
        pltpu.VMEM((INNER, LANE), jnp.int32),
        pltpu.VMEM((CHUNK, HID), jnp.float32),
        pltpu.VMEM((INNER, HID, LANE), jnp.float32),
        pltpu.SemaphoreType.DMA,
    ],
    compiler_params=_SC_PARAMS,
)
def _sc_gather(x_hbm, src_hbm, out_hbm, idx_v, rows_v, tiles_v, sem):
    wid = lax.axis_index("s") * NUM_CORES + lax.axis_index("c")
    iota = lax.iota(jnp.int32, 16)

    def body(i, carry):
        blk = wid * BLOCKS_PER_WORKER + i * INNER
        pltpu.sync_copy(src_hbm.at[pl.ds(blk, INNER)], idx_v)
        cps = [
            pltpu.async_copy(
                x_hbm.at[idx_v.at[j]],
                rows_v.at[pl.ds(j * LANE, LANE)], sem)
            for j in range(INNER)
        ]
        for cp in cps:
            cp.wait()

        def repack(g2, carry2):
            for j in range(INNER):
                r16 = j * LANE + g2 * 16 + iota
                for k in range(HID):
                    vals = plsc.load_gather(
                        rows_v, [r16, jnp.full((16,), k, jnp.int32)])
                    tiles_v[j, k, pl.ds(g2 * 16, 16)] = vals
            return carry2

        lax.fori_loop(0, LANE // 16, repack, 0)
        pltpu.sync_copy(tiles_v, out_hbm.at[pl.ds(blk, INNER)])
        return carry

    lax.fori_loop(0, OUTER, body, 0)


@functools.partial(
    pl.kernel,
    out_type=jax.ShapeDtypeStruct((NUM_CORES, HID, N_ACC), jnp.float32),
    mesh=_MESH,
    scratch_types=[
        pltpu.VMEM((INNER, LANE), jnp.int32),
        pltpu.VMEM((CHUNK, HID), jnp.float32),
        pltpu.VMEM((INNER, HID, LANE), jnp.float32),
        pltpu.VMEM((HID, CHUNK), jnp.float32),
        pltpu.VMEM_SHARED((N_ACC, HID), jnp.float32),
        pltpu.SemaphoreType.DMA,
    ],
    compiler_params=_SC_PARAMS,
)
def _sc_scatter(msg_hbm, dst_hbm, zero_hbm, out_hbm, idx_v, rows_v, tiles_v,
                colsT_v, acc, sem):
    c = lax.axis_index("c")
    s = lax.axis_index("s")
    wid = s * NUM_CORES + c
    t0 = s * ROWS_PER_TILE
    iota = lax.iota(jnp.int32, 16)
    pltpu.sync_copy(zero_hbm.at[pl.ds(t0, ROWS_PER_TILE)],
                    acc.at[pl.ds(t0, ROWS_PER_TILE)])
    plsc.subcore_barrier()

    def body(i, carry):
        blk = wid * BLOCKS_PER_WORKER + i * INNER
        pltpu.sync_copy(dst_hbm.at[pl.ds(blk, INNER)], idx_v)
        pltpu.sync_copy(msg_hbm.at[pl.ds(blk, INNER)], tiles_v)

        def repack(g2, carry2):
            for j in range(INNER):
                r16 = j * LANE + g2 * 16 + iota
                for k in range(HID):
                    vals = tiles_v[j, k, pl.ds(g2 * 16, 16)]
                    plsc.store_scatter(
                        rows_v, [r16, jnp.full((16,), k, jnp.int32)], vals)
            return carry2

        lax.fori_loop(0, LANE // 16, repack, 0)
        for j in range(INNER):
            pltpu.sync_copy(rows_v.at[pl.ds(j * LANE, LANE)],
                            acc.at[idx_v.at[j]], add=True)
        return carry

    lax.fori_loop(0, OUTER, body, 0)
    plsc.subcore_barrier()

    sections = [(q * CHUNK, CHUNK) for q in range(ROWS_PER_TILE // CHUNK)]
    sections.append((ROWS_PER_TILE - ROWS_PER_TILE % CHUNK,
                     ROWS_PER_TILE % CHUNK))

    for off, sz in sections:
        if sz == 0:
            continue
        pltpu.sync_copy(acc.at[pl.ds(t0 + off, sz)],
                        rows_v.at[pl.ds(0, sz)])

        def repackT(g, carry2, sz=sz):
            r16 = g * 16 + iota
            for k in range(HID):
                vals = plsc.load_gather(
                    rows_v, [r16, jnp.full((16,), k, jnp.int32)])
                colsT_v[k, pl.ds(g * 16, 16)] = vals
            return carry2

        lax.fori_loop(0, sz // 16, repackT, 0)
        for k in range(HID):
            pltpu.sync_copy(colsT_v.at[k, pl.ds(0, sz)],
                            out_hbm.at[c, k, pl.ds(t0 + off, sz)])


TB = 100
T_EDGE = TB * LANE


def _edge_body(eat_ref, hs_ref, w1t_ref, b1c_ref, w2t_ref, b2c_ref,
               pt_ref, st_ref, msg_ref):
    f32 = jnp.float32
    hst_in = jnp.transpose(hs_ref[...], (1, 0, 2)).reshape(HID, T_EDGE)
    e1 = jnp.maximum(
        jnp.dot(w1t_ref[...], eat_ref[...], preferred_element_type=f32)
        + b1c_ref[...], 0.0)
    e2 = jnp.maximum(
        jnp.dot(w2t_ref[...], e1, preferred_element_type=f32)
        + b2c_ref[...], 0.0)
    hst = jnp.dot(pt_ref[...], hst_in, preferred_element_type=f32)
    msgT = jnp.dot(st_ref[...], e2 * hst, preferred_element_type=f32)
    msg_ref[...] = jnp.transpose(msgT.reshape(MSG, TB, LANE), (1, 0, 2))


_edge_call = pl.pallas_call(
    _edge_body,
    grid=(E // T_EDGE,),
    in_specs=[
        pl.BlockSpec((D_EDGE, T_EDGE), lambda i: (0, i)),
        pl.BlockSpec((TB, HID, LANE), lambda i: (i, 0, 0)),
        pl.BlockSpec((D_EDGE, D_HID1), lambda i: (0, 0)),
        pl.BlockSpec((D_HID1, 1), lambda i: (0, 0)),
        pl.BlockSpec((MSG * HID, D_HID1), lambda i: (0, 0)),
        pl.BlockSpec((MSG * HID, 1), lambda i: (0, 0)),
        pl.BlockSpec((MSG * HID, HID), lambda i: (0, 0)),
        pl.BlockSpec((MSG, MSG * HID), lambda i: (0, 0)),
    ],
    out_specs=pl.BlockSpec((TB, MSG, LANE), lambda i: (i, 0, 0)),
    out_shape=jax.ShapeDtypeStruct((NUM_BLOCKS, MSG, LANE), jnp.float32),
)


def _gru_body(xt_ref, p_ref, wir, wiz, win, whr, whz, whn,
              bir, biz, bin_, bhr, bhz, bhn, out_ref):
    f32 = jnp.float32
    m = p_ref[0, :, pl.ds(0, N)] + p_ref[1, :, pl.ds(0, N)]
    x = xt_ref[...]
    r = jax.nn.sigmoid(
        jnp.dot(wir[...], m, preferred_element_type=f32) + bir[...]
        + jnp.dot(whr[...], x, preferred_element_type=f32) + bhr[...])
    z = jax.nn.sigmoid(
        jnp.dot(wiz[...], m, preferred_element_type=f32) + biz[...]
        + jnp.dot(whz[...], x, preferred_element_type=f32) + bhz[...])
    n = jnp.tanh(
        jnp.dot(win[...], m, preferred_element_type=f32) + bin_[...]
        + r * (jnp.dot(whn[...], x, preferred_element_type=f32) + bhn[...]))
    out_ref[...] = (1.0 - z) * n + z * x


_wt_spec = pl.BlockSpec((HID, HID), lambda: (0, 0))
_bt_spec = pl.BlockSpec((HID, 1), lambda: (0, 0))
_gru_call = pl.pallas_call(
    _gru_body,
    in_specs=[
        pl.BlockSpec((HID, N), lambda: (0, 0)),
        pl.BlockSpec((NUM_CORES, HID, N_ACC), lambda: (0, 0, 0)),
        _wt_spec, _wt_spec, _wt_spec, _wt_spec, _wt_spec, _wt_spec,
        _bt_spec, _bt_spec, _bt_spec, _bt_spec, _bt_spec, _bt_spec,
    ],
    out_specs=pl.BlockSpec((HID, N), lambda: (0, 0)),
    out_shape=jax.ShapeDtypeStruct((HID, N), jnp.float32),
)


def kernel(x, edge_index, edge_attr, W1, b1, W2, b2, W_ih, b_ih, W_hh, b_hh):
    src = edge_index[0]
    dst = edge_index[1]
    pad = jnp.arange(PAD, dtype=jnp.int32)
    src_p = jnp.concatenate([src, pad]).reshape(NUM_BLOCKS, LANE)
    dst_p = jnp.concatenate(
        [dst, jnp.int32(N) + (pad % N_DUMMY)]).reshape(NUM_BLOCKS, LANE)

    hs3 = _sc_gather(x, src_p)

    P = jnp.tile(jnp.eye(HID, dtype=jnp.float32), (1, MSG))
    S = jnp.repeat(jnp.eye(MSG, dtype=jnp.float32), HID, axis=0)
    msg3 = _edge_call(edge_attr.T, hs3, W1.T, b1.reshape(-1, 1), W2.T,
                      b2.reshape(-1, 1), P.T, S.T)

    zero_acc = jnp.zeros((N_ACC, HID), jnp.float32)
    partialT = _sc_scatter(msg3, dst_p, zero_acc)

    h_newT = _gru_call(
        x.T, partialT,
        W_ih[:, 0:HID].T, W_ih[:, HID:2 * HID].T, W_ih[:, 2 * HID:].T,
        W_hh[:, 0:HID].T, W_hh[:, HID:2 * HID].T, W_hh[:, 2 * HID:].T,
        b_ih[0:HID].reshape(-1, 1), b_ih[HID:2 * HID].reshape(-1, 1),
        b_ih[2 * HID:].reshape(-1, 1),
        b_hh[0:HID].reshape(-1, 1), b_hh[HID:2 * HID].reshape(-1, 1),
        b_hh[2 * HID:].reshape(-1, 1),
    )
    return h_newT.T


# speedup vs baseline: 14.8000x; 1.1169x over previous
import functools

import jax
import jax.numpy as jnp
from jax import lax
from jax.experimental import pallas as pl
from jax.experimental.pallas import tpu as pltpu
from jax.experimental.pallas import tpu_sc as plsc

N = 100000
E = 1600000
D_EDGE = 16
D_HID1 = 16
MSG = 8
HID = 8

NUM_CORES = 2
NUM_SUBCORES = 16
NUM_WORKERS = NUM_CORES * NUM_SUBCORES
LANE = 128
INNER = 8
OUTER = 49
BLOCKS_PER_WORKER = INNER * OUTER
CHUNK = INNER * LANE
E_PAD = NUM_WORKERS * BLOCKS_PER_WORKER * LANE
NUM_BLOCKS = E_PAD // LANE
PAD = E_PAD - E
N_DUMMY = 608
N_ACC = N + N_DUMMY
ROWS_PER_TILE = N_ACC // NUM_SUBCORES

_MESH = plsc.VectorSubcoreMesh(
    core_axis_name="c", subcore_axis_name="s",
    num_cores=NUM_CORES, num_subcores=NUM_SUBCORES)
_SC_PARAMS = pltpu.CompilerParams(
    use_tc_tiling_on_sc=False, needs_layout_passes=False)


@functools.partial(
    pl.kernel,
    out_type=jax.ShapeDtypeStruct((NUM_BLOCKS, HID, LANE), jnp.float32),
    mesh=_MESH,
    scratch_types=[
        pltpu.VMEM((INNER, LANE), jnp.int32),
        pltpu.VMEM((CHUNK, HID), jnp.float32),
        pltpu.VMEM((INNER, HID, LANE), jnp.float32),
        pltpu.VMEM_SHARED((N, HID), jnp.float32),
        pltpu.SemaphoreType.DMA,
    ],
    compiler_params=_SC_PARAMS,
)
def _sc_gather(x_hbm, src_hbm, out_hbm, idx_v, rows_v, tiles_v, xs, sem):
    s = lax.axis_index("s")
    wid = s * NUM_CORES + lax.axis_index("c")
    iota = lax.iota(jnp.int32, 16)
    xrows = N // NUM_SUBCORES
    pltpu.sync_copy(x_hbm.at[pl.ds(s * xrows, xrows)],
                    xs.at[pl.ds(s * xrows, xrows)])
    plsc.subcore_barrier()

    def body(i, carry):
        blk = wid * BLOCKS_PER_WORKER + i * INNER
        pltpu.sync_copy(src_hbm.at[pl.ds(blk, INNER)], idx_v)
        cps = [
            pltpu.async_copy(
                xs.at[idx_v.at[j]],
                rows_v.at[pl.ds(j * LANE, LANE)], sem)
            for j in range(INNER)
        ]
        for cp in cps:
            cp.wait()

        def repack(g2, carry2):
            for j in range(INNER):
                r16 = j * LANE + g2 * 16 + iota
                for k in range(HID):
                    vals = plsc.load_gather(
                        rows_v, [r16, jnp.full((16,), k, jnp.int32)])
                    tiles_v[j, k, pl.ds(g2 * 16, 16)] = vals
            return carry2

        lax.fori_loop(0, LANE // 16, repack, 0)
        pltpu.sync_copy(tiles_v, out_hbm.at[pl.ds(blk, INNER)])
        return carry

    lax.fori_loop(0, OUTER, body, 0)


@functools.partial(
    pl.kernel,
    out_type=jax.ShapeDtypeStruct((NUM_CORES, HID, N_ACC), jnp.float32),
    mesh=_MESH,
    scratch_types=[
        pltpu.VMEM((INNER, LANE), jnp.int32),
        pltpu.VMEM((CHUNK, HID), jnp.float32),
        pltpu.VMEM((INNER, HID, LANE), jnp.float32),
        pltpu.VMEM((HID, CHUNK), jnp.float32),
        pltpu.VMEM_SHARED((N_ACC, HID), jnp.float32),
        pltpu.SemaphoreType.DMA,
    ],
    compiler_params=_SC_PARAMS,
)
def _sc_scatter(msg_hbm, dst_hbm, zero_hbm, out_hbm, idx_v, rows_v, tiles_v,
                colsT_v, acc, sem):
    c = lax.axis_index("c")
    s = lax.axis_index("s")
    wid = s * NUM_CORES + c
    t0 = s * ROWS_PER_TILE
    iota = lax.iota(jnp.int32, 16)
    pltpu.sync_copy(zero_hbm.at[pl.ds(t0, ROWS_PER_TILE)],
                    acc.at[pl.ds(t0, ROWS_PER_TILE)])
    plsc.subcore_barrier()

    def body(i, carry):
        blk = wid * BLOCKS_PER_WORKER + i * INNER
        pltpu.sync_copy(dst_hbm.at[pl.ds(blk, INNER)], idx_v)
        pltpu.sync_copy(msg_hbm.at[pl.ds(blk, INNER)], tiles_v)

        def repack(g2, carry2):
            for j in range(INNER):
                r16 = j * LANE + g2 * 16 + iota
                for k in range(HID):
                    vals = tiles_v[j, k, pl.ds(g2 * 16, 16)]
                    plsc.store_scatter(
                        rows_v, [r16, jnp.full((16,), k, jnp.int32)], vals)
            return carry2

        lax.fori_loop(0, LANE // 16, repack, 0)
        cps = [
            pltpu.async_copy(rows_v.at[pl.ds(j * LANE, LANE)],
                             acc.at[idx_v.at[j]], sem, add=True)
            for j in range(INNER)
        ]
        for cp in cps:
            cp.wait()
        return carry

    lax.fori_loop(0, OUTER, body, 0)
    plsc.subcore_barrier()

    sections = [(q * CHUNK, CHUNK) for q in range(ROWS_PER_TILE // CHUNK)]
    sections.append((ROWS_PER_TILE - ROWS_PER_TILE % CHUNK,
                     ROWS_PER_TILE % CHUNK))

    for off, sz in sections:
        if sz == 0:
            continue
        pltpu.sync_copy(acc.at[pl.ds(t0 + off, sz)],
                        rows_v.at[pl.ds(0, sz)])

        def repackT(g, carry2, sz=sz):
            r16 = g * 16 + iota
            for k in range(HID):
                vals = plsc.load_gather(
                    rows_v, [r16, jnp.full((16,), k, jnp.int32)])
                colsT_v[k, pl.ds(g * 16, 16)] = vals
            return carry2

        lax.fori_loop(0, sz // 16, repackT, 0)
        for k in range(HID):
            pltpu.sync_copy(colsT_v.at[k, pl.ds(0, sz)],
                            out_hbm.at[c, k, pl.ds(t0 + off, sz)])


TB = 100
T_EDGE = TB * LANE


def _edge_body(eat_ref, hs_ref, w1t_ref, b1c_ref, w2t_ref, b2c_ref,
               pt_ref, st_ref, msg_ref):
    f32 = jnp.float32
    hst_in = jnp.transpose(hs_ref[...], (1, 0, 2)).reshape(HID, T_EDGE)
    e1 = jnp.maximum(
        jnp.dot(w1t_ref[...], eat_ref[...], preferred_element_type=f32)
        + b1c_ref[...], 0.0)
    e2 = jnp.maximum(
        jnp.dot(w2t_ref[...], e1, preferred_element_type=f32)
        + b2c_ref[...], 0.0)
    hst = jnp.dot(pt_ref[...], hst_in, preferred_element_type=f32)
    msgT = jnp.dot(st_ref[...], e2 * hst, preferred_element_type=f32)
    msg_ref[...] = jnp.transpose(msgT.reshape(MSG, TB, LANE), (1, 0, 2))


_edge_call = pl.pallas_call(
    _edge_body,
    grid=(E // T_EDGE,),
    in_specs=[
        pl.BlockSpec((D_EDGE, T_EDGE), lambda i: (0, i)),
        pl.BlockSpec((TB, HID, LANE), lambda i: (i, 0, 0)),
        pl.BlockSpec((D_EDGE, D_HID1), lambda i: (0, 0)),
        pl.BlockSpec((D_HID1, 1), lambda i: (0, 0)),
        pl.BlockSpec((MSG * HID, D_HID1), lambda i: (0, 0)),
        pl.BlockSpec((MSG * HID, 1), lambda i: (0, 0)),
        pl.BlockSpec((MSG * HID, HID), lambda i: (0, 0)),
        pl.BlockSpec((MSG, MSG * HID), lambda i: (0, 0)),
    ],
    out_specs=pl.BlockSpec((TB, MSG, LANE), lambda i: (i, 0, 0)),
    out_shape=jax.ShapeDtypeStruct((NUM_BLOCKS, MSG, LANE), jnp.float32),
)


def _gru_body(xt_ref, p_ref, wir, wiz, win, whr, whz, whn,
              bir, biz, bin_, bhr, bhz, bhn, out_ref):
    f32 = jnp.float32
    m = p_ref[0, :, pl.ds(0, N)] + p_ref[1, :, pl.ds(0, N)]
    x = xt_ref[...]
    r = jax.nn.sigmoid(
        jnp.dot(wir[...], m, preferred_element_type=f32) + bir[...]
        + jnp.dot(whr[...], x, preferred_element_type=f32) + bhr[...])
    z = jax.nn.sigmoid(
        jnp.dot(wiz[...], m, preferred_element_type=f32) + biz[...]
        + jnp.dot(whz[...], x, preferred_element_type=f32) + bhz[...])
    n = jnp.tanh(
        jnp.dot(win[...], m, preferred_element_type=f32) + bin_[...]
        + r * (jnp.dot(whn[...], x, preferred_element_type=f32) + bhn[...]))
    out_ref[...] = (1.0 - z) * n + z * x


_wt_spec = pl.BlockSpec((HID, HID), lambda: (0, 0))
_bt_spec = pl.BlockSpec((HID, 1), lambda: (0, 0))
_gru_call = pl.pallas_call(
    _gru_body,
    in_specs=[
        pl.BlockSpec((HID, N), lambda: (0, 0)),
        pl.BlockSpec((NUM_CORES, HID, N_ACC), lambda: (0, 0, 0)),
        _wt_spec, _wt_spec, _wt_spec, _wt_spec, _wt_spec, _wt_spec,
        _bt_spec, _bt_spec, _bt_spec, _bt_spec, _bt_spec, _bt_spec,
    ],
    out_specs=pl.BlockSpec((HID, N), lambda: (0, 0)),
    out_shape=jax.ShapeDtypeStruct((HID, N), jnp.float32),
)


def kernel(x, edge_index, edge_attr, W1, b1, W2, b2, W_ih, b_ih, W_hh, b_hh):
    src = edge_index[0]
    dst = edge_index[1]
    pad = jnp.arange(PAD, dtype=jnp.int32)
    src_p = jnp.concatenate([src, pad]).reshape(NUM_BLOCKS, LANE)
    dst_p = jnp.concatenate(
        [dst, jnp.int32(N) + (pad % N_DUMMY)]).reshape(NUM_BLOCKS, LANE)

    hs3 = _sc_gather(x, src_p)

    P = jnp.tile(jnp.eye(HID, dtype=jnp.float32), (1, MSG))
    S = jnp.repeat(jnp.eye(MSG, dtype=jnp.float32), HID, axis=0)
    msg3 = _edge_call(edge_attr.T, hs3, W1.T, b1.reshape(-1, 1), W2.T,
                      b2.reshape(-1, 1), P.T, S.T)

    zero_acc = jnp.zeros((N_ACC, HID), jnp.float32)
    partialT = _sc_scatter(msg3, dst_p, zero_acc)

    h_newT = _gru_call(
        x.T, partialT,
        W_ih[:, 0:HID].T, W_ih[:, HID:2 * HID].T, W_ih[:, 2 * HID:].T,
        W_hh[:, 0:HID].T, W_hh[:, HID:2 * HID].T, W_hh[:, 2 * HID:].T,
        b_ih[0:HID].reshape(-1, 1), b_ih[HID:2 * HID].reshape(-1, 1),
        b_ih[2 * HID:].reshape(-1, 1),
        b_hh[0:HID].reshape(-1, 1), b_hh[HID:2 * HID].reshape(-1, 1),
        b_hh[2 * HID:].reshape(-1, 1),
    )
    return h_newT.T


# speedup vs baseline: 15.6263x; 1.0558x over previous
import functools

import jax
import jax.numpy as jnp
from jax import lax
from jax.experimental import pallas as pl
from jax.experimental.pallas import tpu as pltpu
from jax.experimental.pallas import tpu_sc as plsc

N = 100000
E = 1600000
D_EDGE = 16
D_HID1 = 16
MSG = 8
HID = 8

NUM_CORES = 2
NUM_SUBCORES = 16
NUM_WORKERS = NUM_CORES * NUM_SUBCORES
LANE = 128
INNER = 14
OUTER = 28
BLOCKS_PER_WORKER = INNER * OUTER
CHUNK = INNER * LANE
E_PAD = NUM_WORKERS * BLOCKS_PER_WORKER * LANE
NUM_BLOCKS = E_PAD // LANE
PAD = E_PAD - E
N_DUMMY = 608
N_ACC = N + N_DUMMY
ROWS_PER_TILE = N_ACC // NUM_SUBCORES

_MESH = plsc.VectorSubcoreMesh(
    core_axis_name="c", subcore_axis_name="s",
    num_cores=NUM_CORES, num_subcores=NUM_SUBCORES)
_SC_PARAMS = pltpu.CompilerParams(
    use_tc_tiling_on_sc=False, needs_layout_passes=False)


@functools.partial(
    pl.kernel,
    out_type=jax.ShapeDtypeStruct((NUM_BLOCKS, HID, LANE), jnp.float32),
    mesh=_MESH,
    scratch_types=[
        pltpu.VMEM((INNER, LANE), jnp.int32),
        pltpu.VMEM((CHUNK, HID), jnp.float32),
        pltpu.VMEM((INNER, HID, LANE), jnp.float32),
        pltpu.VMEM_SHARED((N, HID), jnp.float32),
        pltpu.SemaphoreType.DMA,
    ],
    compiler_params=_SC_PARAMS,
)
def _sc_gather(x_hbm, src_hbm, out_hbm, idx_v, rows_v, tiles_v, xs, sem):
    s = lax.axis_index("s")
    wid = s * NUM_CORES + lax.axis_index("c")
    iota = lax.iota(jnp.int32, 16)
    xrows = N // NUM_SUBCORES
    pltpu.sync_copy(x_hbm.at[pl.ds(s * xrows, xrows)],
                    xs.at[pl.ds(s * xrows, xrows)])
    plsc.subcore_barrier()

    def body(i, carry):
        blk = wid * BLOCKS_PER_WORKER + i * INNER
        pltpu.sync_copy(src_hbm.at[pl.ds(blk, INNER)], idx_v)
        cps = [
            pltpu.async_copy(
                xs.at[idx_v.at[j]],
                rows_v.at[pl.ds(j * LANE, LANE)], sem)
            for j in range(INNER)
        ]
        for cp in cps:
            cp.wait()

        def repack(g2, carry2):
            for j in range(INNER):
                r16 = j * LANE + g2 * 16 + iota
                for k in range(HID):
                    vals = plsc.load_gather(
                        rows_v, [r16, jnp.full((16,), k, jnp.int32)])
                    tiles_v[j, k, pl.ds(g2 * 16, 16)] = vals
            return carry2

        lax.fori_loop(0, LANE // 16, repack, 0)
        pltpu.sync_copy(tiles_v, out_hbm.at[pl.ds(blk, INNER)])
        return carry

    lax.fori_loop(0, OUTER, body, 0)


@functools.partial(
    pl.kernel,
    out_type=jax.ShapeDtypeStruct((NUM_CORES, HID, N_ACC), jnp.float32),
    mesh=_MESH,
    scratch_types=[
        pltpu.VMEM((INNER, LANE), jnp.int32),
        pltpu.VMEM((CHUNK, HID), jnp.float32),
        pltpu.VMEM((INNER, HID, LANE), jnp.float32),
        pltpu.VMEM((HID, CHUNK), jnp.float32),
        pltpu.VMEM_SHARED((N_ACC, HID), jnp.float32),
        pltpu.SemaphoreType.DMA,
    ],
    compiler_params=_SC_PARAMS,
)
def _sc_scatter(msg_hbm, dst_hbm, zero_hbm, out_hbm, idx_v, rows_v, tiles_v,
                colsT_v, acc, sem):
    c = lax.axis_index("c")
    s = lax.axis_index("s")
    wid = s * NUM_CORES + c
    t0 = s * ROWS_PER_TILE
    iota = lax.iota(jnp.int32, 16)
    pltpu.sync_copy(zero_hbm.at[pl.ds(t0, ROWS_PER_TILE)],
                    acc.at[pl.ds(t0, ROWS_PER_TILE)])
    plsc.subcore_barrier()

    def body(i, carry):
        blk = wid * BLOCKS_PER_WORKER + i * INNER
        pltpu.sync_copy(dst_hbm.at[pl.ds(blk, INNER)], idx_v)
        pltpu.sync_copy(msg_hbm.at[pl.ds(blk, INNER)], tiles_v)

        def repack(g2, carry2):
            for j in range(INNER):
                r16 = j * LANE + g2 * 16 + iota
                for k in range(HID):
                    vals = tiles_v[j, k, pl.ds(g2 * 16, 16)]
                    plsc.store_scatter(
                        rows_v, [r16, jnp.full((16,), k, jnp.int32)], vals)
            return carry2

        lax.fori_loop(0, LANE // 16, repack, 0)
        cps = [
            pltpu.async_copy(rows_v.at[pl.ds(j * LANE, LANE)],
                             acc.at[idx_v.at[j]], sem, add=True)
            for j in range(INNER)
        ]
        for cp in cps:
            cp.wait()
        return carry

    lax.fori_loop(0, OUTER, body, 0)
    plsc.subcore_barrier()

    sections = [(q * CHUNK, CHUNK) for q in range(ROWS_PER_TILE // CHUNK)]
    sections.append((ROWS_PER_TILE - ROWS_PER_TILE % CHUNK,
                     ROWS_PER_TILE % CHUNK))

    for off, sz in sections:
        if sz == 0:
            continue
        pltpu.sync_copy(acc.at[pl.ds(t0 + off, sz)],
                        rows_v.at[pl.ds(0, sz)])

        def repackT(g, carry2, sz=sz):
            r16 = g * 16 + iota
            for k in range(HID):
                vals = plsc.load_gather(
                    rows_v, [r16, jnp.full((16,), k, jnp.int32)])
                colsT_v[k, pl.ds(g * 16, 16)] = vals
            return carry2

        lax.fori_loop(0, sz // 16, repackT, 0)
        for k in range(HID):
            pltpu.sync_copy(colsT_v.at[k, pl.ds(0, sz)],
                            out_hbm.at[c, k, pl.ds(t0 + off, sz)])


TB = 100
T_EDGE = TB * LANE


def _edge_body(eat_ref, hs_ref, w1t_ref, b1c_ref, w2t_ref, b2c_ref,
               pt_ref, st_ref, msg_ref):
    f32 = jnp.float32
    hst_in = jnp.transpose(hs_ref[...], (1, 0, 2)).reshape(HID, T_EDGE)
    e1 = jnp.maximum(
        jnp.dot(w1t_ref[...], eat_ref[...], preferred_element_type=f32)
        + b1c_ref[...], 0.0)
    e2 = jnp.maximum(
        jnp.dot(w2t_ref[...], e1, preferred_element_type=f32)
        + b2c_ref[...], 0.0)
    hst = jnp.dot(pt_ref[...], hst_in, preferred_element_type=f32)
    msgT = jnp.dot(st_ref[...], e2 * hst, preferred_element_type=f32)
    msg_ref[...] = jnp.transpose(msgT.reshape(MSG, TB, LANE), (1, 0, 2))


_edge_call = pl.pallas_call(
    _edge_body,
    grid=(E // T_EDGE,),
    in_specs=[
        pl.BlockSpec((D_EDGE, T_EDGE), lambda i: (0, i)),
        pl.BlockSpec((TB, HID, LANE), lambda i: (i, 0, 0)),
        pl.BlockSpec((D_EDGE, D_HID1), lambda i: (0, 0)),
        pl.BlockSpec((D_HID1, 1), lambda i: (0, 0)),
        pl.BlockSpec((MSG * HID, D_HID1), lambda i: (0, 0)),
        pl.BlockSpec((MSG * HID, 1), lambda i: (0, 0)),
        pl.BlockSpec((MSG * HID, HID), lambda i: (0, 0)),
        pl.BlockSpec((MSG, MSG * HID), lambda i: (0, 0)),
    ],
    out_specs=pl.BlockSpec((TB, MSG, LANE), lambda i: (i, 0, 0)),
    out_shape=jax.ShapeDtypeStruct((NUM_BLOCKS, MSG, LANE), jnp.float32),
)


def _gru_body(xt_ref, p_ref, wir, wiz, win, whr, whz, whn,
              bir, biz, bin_, bhr, bhz, bhn, out_ref):
    f32 = jnp.float32
    m = p_ref[0, :, pl.ds(0, N)] + p_ref[1, :, pl.ds(0, N)]
    x = xt_ref[...]
    r = jax.nn.sigmoid(
        jnp.dot(wir[...], m, preferred_element_type=f32) + bir[...]
        + jnp.dot(whr[...], x, preferred_element_type=f32) + bhr[...])
    z = jax.nn.sigmoid(
        jnp.dot(wiz[...], m, preferred_element_type=f32) + biz[...]
        + jnp.dot(whz[...], x, preferred_element_type=f32) + bhz[...])
    n = jnp.tanh(
        jnp.dot(win[...], m, preferred_element_type=f32) + bin_[...]
        + r * (jnp.dot(whn[...], x, preferred_element_type=f32) + bhn[...]))
    out_ref[...] = (1.0 - z) * n + z * x


_wt_spec = pl.BlockSpec((HID, HID), lambda: (0, 0))
_bt_spec = pl.BlockSpec((HID, 1), lambda: (0, 0))
_gru_call = pl.pallas_call(
    _gru_body,
    in_specs=[
        pl.BlockSpec((HID, N), lambda: (0, 0)),
        pl.BlockSpec((NUM_CORES, HID, N_ACC), lambda: (0, 0, 0)),
        _wt_spec, _wt_spec, _wt_spec, _wt_spec, _wt_spec, _wt_spec,
        _bt_spec, _bt_spec, _bt_spec, _bt_spec, _bt_spec, _bt_spec,
    ],
    out_specs=pl.BlockSpec((HID, N), lambda: (0, 0)),
    out_shape=jax.ShapeDtypeStruct((HID, N), jnp.float32),
)


def kernel(x, edge_index, edge_attr, W1, b1, W2, b2, W_ih, b_ih, W_hh, b_hh):
    src = edge_index[0]
    dst = edge_index[1]
    pad = jnp.arange(PAD, dtype=jnp.int32)
    src_p = jnp.concatenate([src, pad]).reshape(NUM_BLOCKS, LANE)
    dst_p = jnp.concatenate(
        [dst, jnp.int32(N) + (pad % N_DUMMY)]).reshape(NUM_BLOCKS, LANE)

    hs3 = _sc_gather(x, src_p)

    P = jnp.tile(jnp.eye(HID, dtype=jnp.float32), (1, MSG))
    S = jnp.repeat(jnp.eye(MSG, dtype=jnp.float32), HID, axis=0)
    msg3 = _edge_call(edge_attr.T, hs3, W1.T, b1.reshape(-1, 1), W2.T,
                      b2.reshape(-1, 1), P.T, S.T)

    zero_acc = jnp.zeros((N_ACC, HID), jnp.float32)
    partialT = _sc_scatter(msg3, dst_p, zero_acc)

    h_newT = _gru_call(
        x.T, partialT,
        W_ih[:, 0:HID].T, W_ih[:, HID:2 * HID].T, W_ih[:, 2 * HID:].T,
        W_hh[:, 0:HID].T, W_hh[:, HID:2 * HID].T, W_hh[:, 2 * HID:].T,
        b_ih[0:HID].reshape(-1, 1), b_ih[HID:2 * HID].reshape(-1, 1),
        b_ih[2 * HID:].reshape(-1, 1),
        b_hh[0:HID].reshape(-1, 1), b_hh[HID:2 * HID].reshape(-1, 1),
        b_hh[2 * HID:].reshape(-1, 1),
    )
    return h_newT.T


# speedup vs baseline: 15.8667x; 1.0154x over previous
import functools

import jax
import jax.numpy as jnp
from jax import lax
from jax.experimental import pallas as pl
from jax.experimental.pallas import tpu as pltpu
from jax.experimental.pallas import tpu_sc as plsc

N = 100000
E = 1600000
D_EDGE = 16
D_HID1 = 16
MSG = 8
HID = 8

NUM_CORES = 2
NUM_SUBCORES = 16
NUM_WORKERS = NUM_CORES * NUM_SUBCORES
LANE = 128
INNER = 14
OUTER = 28
BLOCKS_PER_WORKER = INNER * OUTER
CHUNK = INNER * LANE
E_PAD = NUM_WORKERS * BLOCKS_PER_WORKER * LANE
NUM_BLOCKS = E_PAD // LANE
PAD = E_PAD - E
N_DUMMY = 608
N_ACC = N + N_DUMMY
ROWS_PER_TILE = N_ACC // NUM_SUBCORES

_MESH = plsc.VectorSubcoreMesh(
    core_axis_name="c", subcore_axis_name="s",
    num_cores=NUM_CORES, num_subcores=NUM_SUBCORES)
_SC_PARAMS = pltpu.CompilerParams(
    use_tc_tiling_on_sc=False, needs_layout_passes=False)


@functools.partial(
    pl.kernel,
    out_type=jax.ShapeDtypeStruct((NUM_BLOCKS, HID, LANE), jnp.float32),
    mesh=_MESH,
    scratch_types=[
        pltpu.VMEM((INNER, LANE), jnp.int32),
        pltpu.VMEM((CHUNK, HID), jnp.float32),
        pltpu.VMEM((INNER, HID, LANE), jnp.float32),
        pltpu.VMEM_SHARED((N, HID), jnp.float32),
        pltpu.SemaphoreType.DMA,
    ],
    compiler_params=_SC_PARAMS,
)
def _sc_gather(x_hbm, src_hbm, out_hbm, idx_v, rows_v, tiles_v, xs, sem):
    s = lax.axis_index("s")
    wid = s * NUM_CORES + lax.axis_index("c")
    iota = lax.iota(jnp.int32, 16)
    xrows = N // NUM_SUBCORES
    pltpu.sync_copy(x_hbm.at[pl.ds(s * xrows, xrows)],
                    xs.at[pl.ds(s * xrows, xrows)])
    plsc.subcore_barrier()

    def body(i, carry):
        blk = wid * BLOCKS_PER_WORKER + i * INNER
        pltpu.sync_copy(src_hbm.at[pl.ds(blk, INNER)], idx_v)
        cps = [
            pltpu.async_copy(
                xs.at[idx_v.at[j]],
                rows_v.at[pl.ds(j * LANE, LANE)], sem)
            for j in range(INNER)
        ]
        for cp in cps:
            cp.wait()

        def repack(g2, carry2):
            for j in range(INNER):
                r16 = j * LANE + g2 * 16 + iota
                for k in range(HID):
                    vals = plsc.load_gather(
                        rows_v, [r16, jnp.full((16,), k, jnp.int32)])
                    tiles_v[j, k, pl.ds(g2 * 16, 16)] = vals
            return carry2

        lax.fori_loop(0, LANE // 16, repack, 0)
        pltpu.sync_copy(tiles_v, out_hbm.at[pl.ds(blk, INNER)])
        return carry

    lax.fori_loop(0, OUTER, body, 0)


@functools.partial(
    pl.kernel,
    out_type=jax.ShapeDtypeStruct((NUM_CORES, HID, N_ACC), jnp.float32),
    mesh=_MESH,
    scratch_types=[
        pltpu.VMEM((INNER, LANE), jnp.int32),
        pltpu.VMEM((CHUNK, HID), jnp.float32),
        pltpu.VMEM((INNER, HID, LANE), jnp.float32),
        pltpu.VMEM((HID, CHUNK), jnp.float32),
        pltpu.VMEM_SHARED((N_ACC, HID), jnp.float32),
        pltpu.SemaphoreType.DMA,
    ],
    compiler_params=_SC_PARAMS,
)
def _sc_scatter(msg_hbm, dst_hbm, zero_hbm, out_hbm, idx_v, rows_v, tiles_v,
                colsT_v, acc, sem):
    c = lax.axis_index("c")
    s = lax.axis_index("s")
    wid = s * NUM_CORES + c
    t0 = s * ROWS_PER_TILE
    iota = lax.iota(jnp.int32, 16)
    pltpu.sync_copy(zero_hbm.at[pl.ds(t0, ROWS_PER_TILE)],
                    acc.at[pl.ds(t0, ROWS_PER_TILE)])
    plsc.subcore_barrier()

    def body(i, carry):
        blk = wid * BLOCKS_PER_WORKER + i * INNER
        pltpu.sync_copy(dst_hbm.at[pl.ds(blk, INNER)], idx_v)
        pltpu.sync_copy(msg_hbm.at[pl.ds(blk, INNER)], tiles_v)

        def repack(g2, carry2):
            for j in range(INNER):
                r16 = j * LANE + g2 * 16 + iota
                for k in range(HID):
                    vals = tiles_v[j, k, pl.ds(g2 * 16, 16)]
                    plsc.store_scatter(
                        rows_v, [r16, jnp.full((16,), k, jnp.int32)], vals)
            return carry2

        lax.fori_loop(0, LANE // 16, repack, 0)
        cps = [
            pltpu.async_copy(rows_v.at[pl.ds(j * LANE, LANE)],
                             acc.at[idx_v.at[j]], sem, add=True)
            for j in range(INNER)
        ]
        for cp in cps:
            cp.wait()
        return carry

    lax.fori_loop(0, OUTER, body, 0)
    plsc.subcore_barrier()

    sections = [(q * CHUNK, CHUNK) for q in range(ROWS_PER_TILE // CHUNK)]
    sections.append((ROWS_PER_TILE - ROWS_PER_TILE % CHUNK,
                     ROWS_PER_TILE % CHUNK))

    for off, sz in sections:
        if sz == 0:
            continue
        pltpu.sync_copy(acc.at[pl.ds(t0 + off, sz)],
                        rows_v.at[pl.ds(0, sz)])

        def repackT(g, carry2, sz=sz):
            r16 = g * 16 + iota
            for k in range(HID):
                vals = plsc.load_gather(
                    rows_v, [r16, jnp.full((16,), k, jnp.int32)])
                colsT_v[k, pl.ds(g * 16, 16)] = vals
            return carry2

        lax.fori_loop(0, sz // 16, repackT, 0)
        for k in range(HID):
            pltpu.sync_copy(colsT_v.at[k, pl.ds(0, sz)],
                            out_hbm.at[c, k, pl.ds(t0 + off, sz)])


TB = 250
T_EDGE = TB * LANE


def _edge_body(eat_ref, hs_ref, w1t_ref, b1c_ref, w2t_ref, b2c_ref,
               pt_ref, st_ref, msg_ref):
    f32 = jnp.float32
    hst_in = jnp.transpose(hs_ref[...], (1, 0, 2)).reshape(HID, T_EDGE)
    e1 = jnp.maximum(
        jnp.dot(w1t_ref[...], eat_ref[...], preferred_element_type=f32)
        + b1c_ref[...], 0.0)
    e2 = jnp.maximum(
        jnp.dot(w2t_ref[...], e1, preferred_element_type=f32)
        + b2c_ref[...], 0.0)
    hst = jnp.dot(pt_ref[...], hst_in, preferred_element_type=f32)
    msgT = jnp.dot(st_ref[...], e2 * hst, preferred_element_type=f32)
    msg_ref[...] = jnp.transpose(msgT.reshape(MSG, TB, LANE), (1, 0, 2))


_edge_call = pl.pallas_call(
    _edge_body,
    grid=(E // T_EDGE,),
    in_specs=[
        pl.BlockSpec((D_EDGE, T_EDGE), lambda i: (0, i)),
        pl.BlockSpec((TB, HID, LANE), lambda i: (i, 0, 0)),
        pl.BlockSpec((D_EDGE, D_HID1), lambda i: (0, 0)),
        pl.BlockSpec((D_HID1, 1), lambda i: (0, 0)),
        pl.BlockSpec((MSG * HID, D_HID1), lambda i: (0, 0)),
        pl.BlockSpec((MSG * HID, 1), lambda i: (0, 0)),
        pl.BlockSpec((MSG * HID, HID), lambda i: (0, 0)),
        pl.BlockSpec((MSG, MSG * HID), lambda i: (0, 0)),
    ],
    out_specs=pl.BlockSpec((TB, MSG, LANE), lambda i: (i, 0, 0)),
    out_shape=jax.ShapeDtypeStruct((NUM_BLOCKS, MSG, LANE), jnp.float32),
)


def _gru_body(xt_ref, p_ref, wir, wiz, win, whr, whz, whn,
              bir, biz, bin_, bhr, bhz, bhn, out_ref):
    f32 = jnp.float32
    m = p_ref[0, :, pl.ds(0, N)] + p_ref[1, :, pl.ds(0, N)]
    x = xt_ref[...]
    r = jax.nn.sigmoid(
        jnp.dot(wir[...], m, preferred_element_type=f32) + bir[...]
        + jnp.dot(whr[...], x, preferred_element_type=f32) + bhr[...])
    z = jax.nn.sigmoid(
        jnp.dot(wiz[...], m, preferred_element_type=f32) + biz[...]
        + jnp.dot(whz[...], x, preferred_element_type=f32) + bhz[...])
    n = jnp.tanh(
        jnp.dot(win[...], m, preferred_element_type=f32) + bin_[...]
        + r * (jnp.dot(whn[...], x, preferred_element_type=f32) + bhn[...]))
    out_ref[...] = (1.0 - z) * n + z * x


_wt_spec = pl.BlockSpec((HID, HID), lambda: (0, 0))
_bt_spec = pl.BlockSpec((HID, 1), lambda: (0, 0))
_gru_call = pl.pallas_call(
    _gru_body,
    in_specs=[
        pl.BlockSpec((HID, N), lambda: (0, 0)),
        pl.BlockSpec((NUM_CORES, HID, N_ACC), lambda: (0, 0, 0)),
        _wt_spec, _wt_spec, _wt_spec, _wt_spec, _wt_spec, _wt_spec,
        _bt_spec, _bt_spec, _bt_spec, _bt_spec, _bt_spec, _bt_spec,
    ],
    out_specs=pl.BlockSpec((HID, N), lambda: (0, 0)),
    out_shape=jax.ShapeDtypeStruct((HID, N), jnp.float32),
)


def kernel(x, edge_index, edge_attr, W1, b1, W2, b2, W_ih, b_ih, W_hh, b_hh):
    src = edge_index[0]
    dst = edge_index[1]
    pad = jnp.arange(PAD, dtype=jnp.int32)
    src_p = jnp.concatenate([src, pad]).reshape(NUM_BLOCKS, LANE)
    dst_p = jnp.concatenate(
        [dst, jnp.int32(N) + (pad % N_DUMMY)]).reshape(NUM_BLOCKS, LANE)

    hs3 = _sc_gather(x, src_p)

    P = jnp.tile(jnp.eye(HID, dtype=jnp.float32), (1, MSG))
    S = jnp.repeat(jnp.eye(MSG, dtype=jnp.float32), HID, axis=0)
    msg3 = _edge_call(edge_attr.T, hs3, W1.T, b1.reshape(-1, 1), W2.T,
                      b2.reshape(-1, 1), P.T, S.T)

    zero_acc = jnp.zeros((N_ACC, HID), jnp.float32)
    partialT = _sc_scatter(msg3, dst_p, zero_acc)

    h_newT = _gru_call(
        x.T, partialT,
        W_ih[:, 0:HID].T, W_ih[:, HID:2 * HID].T, W_ih[:, 2 * HID:].T,
        W_hh[:, 0:HID].T, W_hh[:, HID:2 * HID].T, W_hh[:, 2 * HID:].T,
        b_ih[0:HID].reshape(-1, 1), b_ih[HID:2 * HID].reshape(-1, 1),
        b_ih[2 * HID:].reshape(-1, 1),
        b_hh[0:HID].reshape(-1, 1), b_hh[HID:2 * HID].reshape(-1, 1),
        b_hh[2 * HID:].reshape(-1, 1),
    )
    return h_newT.T
